# baseline (speedup 1.0000x reference)
import functools
import math

import jax
import jax.numpy as jnp
from jax import lax
from jax.experimental import pallas as pl
from jax.experimental.pallas import tpu as pltpu


B = 2
T = 4096
HD = 2048
NH = 16
KVH = 8
D = 128
SINK = 128
WIN = 2048
M = 64
G = 2
N = 128
DI = HD
K = 4
CONV = DI + 2 * G * N
FF = 8192
E = T - WIN - SINK
T2 = SINK + M + WIN
T2P = 2304
NKV = 2 * G * N
CIN = DI + CONV + NH
CINP = 4864
L = 128
NC = E // L
EPS = 1e-6
SCALE = 1.0 / math.sqrt(D)
NEG = -1e30

f32 = jnp.float32
bf16 = jnp.bfloat16


def _rms(xv, w):
    ms = jnp.mean(xv * xv, axis=-1, keepdims=True)
    return xv * lax.rsqrt(ms + EPS) * w


def _softplus(xv):
    return jnp.maximum(xv, 0.0) + jnp.log1p(jnp.exp(-jnp.abs(xv)))


def _silu(xv):
    return xv * jax.nn.sigmoid(xv)


def _rope(xv, c, s, groups):
    parts = []
    for g in range(groups):
        a = xv[:, g * 128:g * 128 + 64]
        b = xv[:, g * 128 + 64:(g + 1) * 128]
        parts.append(-b)
        parts.append(a)
    rh = jnp.concatenate(parts, axis=1)
    return xv * c + rh * s


def _dot(a, b):
    return jnp.dot(a, b, preferred_element_type=f32)


def _dotg(a, b, dims):
    return lax.dot_general(a, b, (dims, ((), ())), preferred_element_type=f32)


def _inproj_body(x_ref, w_ref, lw_ref, zx_ref, xn_s):
    j = pl.program_id(2)

    @pl.when(j == 0)
    def _():
        xn = _rms(x_ref[0], lw_ref[...])
        xn_s[...] = xn.astype(bf16)

    zx_ref[:, 0, :] = _dot(xn_s[...], w_ref[...]).astype(bf16)


def _inproj(xm, w_in_bf, ln1w):
    return pl.pallas_call(
        _inproj_body,
        grid=(B, 2, CINP // 256),
        in_specs=[
            pl.BlockSpec((1, E // 2, HD), lambda b, p, j: (b, p, 0)),
            pl.BlockSpec((HD, 256), lambda b, p, j: (0, j)),
            pl.BlockSpec((1, HD), lambda b, p, j: (0, 0)),
        ],
        out_specs=pl.BlockSpec(
            (E // 2, 1, 256), lambda b, p, j: (p, 0, b * (CINP // 256) + j)),
        out_shape=jax.ShapeDtypeStruct((E, 1, B * CINP), bf16),
        scratch_shapes=[pltpu.VMEM((E // 2, HD), bf16)],
        compiler_params=pltpu.CompilerParams(
            dimension_semantics=("parallel", "arbitrary", "arbitrary"),
            vmem_limit_bytes=48 * 1024 * 1024),
        name="inproj",
    )(xm, w_in_bf, ln1w)


def _dtproj_body(x_ref, lw_ref, w16_ref, dtb_ref, ae_ref, dt_ref, la_ref):
    xn = _rms(x_ref[0], lw_ref[...]).astype(bf16)
    raw = _dotg(w16_ref[...], xn, (((0,), (1,))))
    dt = _softplus(raw + dtb_ref[...])
    la = -ae_ref[...] * dt
    dt_ref[...] = dt.reshape(NH, 1, L)
    la_ref[...] = la.reshape(NH, 1, L)


def _dtproj(x, ln1w, w16_bf, dtb_col, aexp_col):
    return pl.pallas_call(
        _dtproj_body,
        grid=(B, NC),
        in_specs=[
            pl.BlockSpec((1, L, HD), lambda b, c: (b, c + 1, 0)),
            pl.BlockSpec((1, HD), lambda b, c: (0, 0)),
            pl.BlockSpec((HD, NH), lambda b, c: (0, 0)),
            pl.BlockSpec((NH, 1), lambda b, c: (0, 0)),
            pl.BlockSpec((NH, 1), lambda b, c: (0, 0)),
        ],
        out_specs=[
            pl.BlockSpec((NH, 1, L), lambda b, c: (b, 0, c)),
            pl.BlockSpec((NH, 1, L), lambda b, c: (b, 0, c)),
        ],
        out_shape=[
            jax.ShapeDtypeStruct((B * NH, 1, E), f32),
            jax.ShapeDtypeStruct((B * NH, 1, E), f32),
        ],
        compiler_params=pltpu.CompilerParams(
            dimension_semantics=("parallel", "arbitrary")),
        name="dtproj",
    )(x, ln1w, w16_bf, dtb_col, aexp_col)


def _conv_body(xbc_ref, wc_ref, cb_ref, out_ref):
    xv = xbc_ref[:, 0, :].astype(f32)
    acc = cb_ref[...] + wc_ref[3:4, :] * xv
    for k in range(3):
        sh = 3 - k
        shifted = jnp.concatenate(
            [jnp.zeros((sh, 128), f32), xv[:E - sh, :]], axis=0)
        acc = acc + wc_ref[k:k + 1, :] * shifted
    out_ref[:, 0, :] = _silu(acc).astype(bf16)


def _conv(zx, wconv, convb):
    return pl.pallas_call(
        _conv_body,
        grid=(B, CONV // 128),
        in_specs=[
            pl.BlockSpec((E, 1, 128),
                         lambda b, j: (0, 0, b * (CINP // 128) + DI // 128 + j)),
            pl.BlockSpec((K, 128), lambda b, j: (0, j)),
            pl.BlockSpec((1, 128), lambda b, j: (0, j)),
        ],
        out_specs=pl.BlockSpec((E, 1, 128), lambda b, j: (0, 0, b * (CONV // 128) + j)),
        out_shape=jax.ShapeDtypeStruct((E, 1, B * CONV), bf16),
        compiler_params=pltpu.CompilerParams(
            dimension_semantics=("parallel", "arbitrary")),
        name="conv",
    )(zx, wconv, convb)


def _ssd_body(xbc_ref, dt_ref, la_ref, mD_ref, y_ref, st_ref):
    c = pl.program_id(1)

    @pl.when(c == 0)
    def _():
        st_ref[...] = jnp.zeros_like(st_ref)

    la_all = la_ref[:, 0, :]
    dt_all = dt_ref[:, 0, :]
    s_all = la_all
    sh = 1
    while sh < L:
        s_all = s_all + jnp.concatenate(
            [jnp.zeros((NH, sh), f32), s_all[:, :L - sh]], axis=1)
        sh *= 2
    sT = jnp.transpose(s_all)
    dtT = jnp.transpose(dt_all)

    xbc = xbc_ref[:, 0, :]
    row_i = lax.broadcasted_iota(jnp.int32, (L, L), 0)
    col_i = lax.broadcasted_iota(jnp.int32, (L, L), 1)
    tri = row_i >= col_i

    for h in range(NH):
        g = h // (NH // G)
        xs = xbc[:, h * 128:(h + 1) * 128]
        Bm = xbc[:, DI + g * 128:DI + (g + 1) * 128]
        Cm = xbc[:, DI + G * 128 + g * 128:DI + G * 128 + (g + 1) * 128]
        xs_f = xs.astype(f32)
        s_row = s_all[h:h + 1, :]
        s_colb = jnp.broadcast_to(sT[:, h:h + 1], (L, L))
        dt_row = dt_all[h:h + 1, :]
        arg = jnp.where(tri, s_colb - jnp.broadcast_to(s_row, (L, L)), NEG)
        Wm = jnp.exp(arg) * jnp.broadcast_to(dt_row, (L, L))
        CB = _dotg(Cm, Bm, ((1,), (1,)))
        scores = CB * Wm
        h0 = st_ref[h * 128:(h + 1) * 128, :]
        Ch0 = _dotg(Cm, h0.astype(bf16), ((1,), (1,)))
        exp_s = jnp.exp(s_colb)
        y_h = (_dotg(scores.astype(bf16), xs, ((1,), (0,)))
               + Ch0 * exp_s + mD_ref[h] * xs_f)
        y_ref[:, 0, h * 128:(h + 1) * 128] = y_h.astype(bf16)
        s_last = sT[L - 1, h]
        w_col = jnp.exp(s_last - sT[:, h:h + 1]) * dtT[:, h:h + 1]
        xs_sc = xs_f * jnp.broadcast_to(w_col, (L, L))
        st_ref[h * 128:(h + 1) * 128, :] = (
            h0 * jnp.exp(s_last)
            + _dotg(xs_sc.astype(bf16), Bm, ((0,), (0,))))


def _ssd(xbca, dt_t, la_t, mD):
    return pl.pallas_call(
        _ssd_body,
        grid=(B, NC),
        in_specs=[
            pl.BlockSpec((L, 1, CONV), lambda b, c: (c, 0, b)),
            pl.BlockSpec((NH, 1, L), lambda b, c: (b, 0, c)),
            pl.BlockSpec((NH, 1, L), lambda b, c: (b, 0, c)),
            pl.BlockSpec(memory_space=pltpu.SMEM),
        ],
        out_specs=pl.BlockSpec((L, 1, DI), lambda b, c: (c, 0, b)),
        out_shape=jax.ShapeDtypeStruct((E, 1, B * DI), bf16),
        scratch_shapes=[pltpu.VMEM((NH * D, N), f32)],
        compiler_params=pltpu.CompilerParams(
            dimension_semantics=("parallel", "arbitrary")),
        name="ssd",
    )(xbca, dt_t, la_t, mD)


def _gather_body(idx_ref, y_ref, zx_ref, ym_ref, zm_ref):
    ym_ref[...] = y_ref[...]
    zm_ref[...] = zx_ref[...]


def _gather(mem_idx, y, zx):
    return pl.pallas_call(
        _gather_body,
        grid_spec=pltpu.PrefetchScalarGridSpec(
            num_scalar_prefetch=1,
            grid=(M,),
            in_specs=[
                pl.BlockSpec((1, 1, B * DI), lambda m, idx: (idx[m], 0, 0)),
                pl.BlockSpec((1, 1, B * CINP), lambda m, idx: (idx[m], 0, 0)),
            ],
            out_specs=[
                pl.BlockSpec((1, 1, B * DI), lambda m, idx: (m, 0, 0)),
                pl.BlockSpec((1, 1, B * CINP), lambda m, idx: (m, 0, 0)),
            ],
        ),
        out_shape=[
            jax.ShapeDtypeStruct((M, 1, B * DI), bf16),
            jax.ShapeDtypeStruct((M, 1, B * CINP), bf16),
        ],
        name="gather",
    )(mem_idx, y, zx)


def _memfin_body(ym_ref, zm_ref, nw_ref, mow_ref, wkv_ref, bkv_ref,
                 c8_ref, s8_ref, mk_ref, mv_ref):
    y64 = ym_ref[:, 0, :].astype(f32)
    z64 = zm_ref[:, 0, :DI].astype(f32)
    gt = y64 * _silu(z64)
    nr = _rms(gt, nw_ref[...]).astype(bf16)
    act = _dot(nr, mow_ref[...]).astype(bf16)
    kv = _dot(act, wkv_ref[...]) + bkv_ref[...]
    kpart = kv[:, :KVH * D]
    mk = _rope(kpart, c8_ref[...], s8_ref[...], KVH)
    mk_ref[0] = mk.astype(bf16)
    mv_ref[0] = kv[:, KVH * D:].astype(bf16)


def _memfin(ym, zm, mnorm, mow_bf, wkv_bf, bkv, cos8, sin8):
    return pl.pallas_call(
        _memfin_body,
        grid=(B,),
        in_specs=[
            pl.BlockSpec((M, 1, DI), lambda b: (0, 0, b)),
            pl.BlockSpec((M, 1, CINP), lambda b: (0, 0, b)),
            pl.BlockSpec((1, DI), lambda b: (0, 0)),
            pl.BlockSpec((DI, DI), lambda b: (0, 0)),
            pl.BlockSpec((DI, 2 * KVH * D), lambda b: (0, 0)),
            pl.BlockSpec((1, 2 * KVH * D), lambda b: (0, 0)),
            pl.BlockSpec((M, KVH * D), lambda b: (0, 0)),
            pl.BlockSpec((M, KVH * D), lambda b: (0, 0)),
        ],
        out_specs=[
            pl.BlockSpec((1, M, KVH * D), lambda b: (b, 0, 0)),
            pl.BlockSpec((1, M, KVH * D), lambda b: (b, 0, 0)),
        ],
        out_shape=[
            jax.ShapeDtypeStruct((B, M, KVH * D), bf16),
            jax.ShapeDtypeStruct((B, M, KVH * D), bf16),
        ],
        compiler_params=pltpu.CompilerParams(
            dimension_semantics=("parallel",),
            vmem_limit_bytes=48 * 1024 * 1024),
        name="memfin",
    )(ym, zm, mnorm, mow_bf, wkv_bf, bkv, cos8, sin8)


def _qkv_body(x_ref, lw_ref, w_ref, bias_ref, c_ref, s_ref,
              q_ref, k_ref, v_ref):
    xn = _rms(x_ref[0], lw_ref[...]).astype(bf16)
    qkv = _dot(xn, w_ref[...]) + bias_ref[...]
    q = qkv[:, :NH * D]
    k = qkv[:, NH * D:NH * D + KVH * D]
    v = qkv[:, NH * D + KVH * D:]
    cos1 = c_ref[0]
    sin1 = s_ref[0]
    c16 = jnp.concatenate([cos1] * NH, axis=1)
    s16 = jnp.concatenate([sin1] * NH, axis=1)
    c8 = jnp.concatenate([cos1] * KVH, axis=1)
    s8 = jnp.concatenate([sin1] * KVH, axis=1)
    q_ref[0] = (_rope(q, c16, s16, NH) * SCALE).astype(bf16)
    k_ref[0] = _rope(k, c8, s8, KVH).astype(bf16)
    v_ref[0] = v.astype(bf16)


def _qkv(x2p, ln1w, wqkv_bf, bqkv, cos2p, sin2p):
    return pl.pallas_call(
        _qkv_body,
        grid=(B, T2P // 128),
        in_specs=[
            pl.BlockSpec((1, 128, HD), lambda b, r: (b, r, 0)),
            pl.BlockSpec((1, HD), lambda b, r: (0, 0)),
            pl.BlockSpec((HD, NH * D + 2 * KVH * D), lambda b, r: (0, 0)),
            pl.BlockSpec((1, NH * D + 2 * KVH * D), lambda b, r: (0, 0)),
            pl.BlockSpec((1, 128, D), lambda b, r: (b, r, 0)),
            pl.BlockSpec((1, 128, D), lambda b, r: (b, r, 0)),
        ],
        out_specs=[
            pl.BlockSpec((1, 128, NH * D), lambda b, r: (b, r, 0)),
            pl.BlockSpec((1, 128, KVH * D), lambda b, r: (b, r, 0)),
            pl.BlockSpec((1, 128, KVH * D), lambda b, r: (b, r, 0)),
        ],
        out_shape=[
            jax.ShapeDtypeStruct((B, T2P, NH * D), bf16),
            jax.ShapeDtypeStruct((B, T2P, KVH * D), bf16),
            jax.ShapeDtypeStruct((B, T2P, KVH * D), bf16),
        ],
        compiler_params=pltpu.CompilerParams(
            dimension_semantics=("parallel", "arbitrary"),
            vmem_limit_bytes=56 * 1024 * 1024),
        name="qkv",
    )(x2p, ln1w, wqkv_bf, bqkv, cos2p, sin2p)


BQ = 256
NQB = T2P // BQ


KVB = 768
NKVB = T2P // KVB
QPC = KVB // BQ


def _attn_body(q_ref, k_ref, v_ref, o_ref, m_s, l_s, acc_s):
    qi = pl.program_id(2)
    m_s[...] = jnp.full_like(m_s, NEG)
    l_s[...] = jnp.zeros_like(l_s)
    acc_s[...] = jnp.zeros_like(acc_s)
    qb = q_ref[0]

    def _chunk(j, masked):
        kb = k_ref[0, j * KVB:(j + 1) * KVB, :]
        s = _dotg(qb, kb, ((1,), (1,)))
        if masked:
            rows = qi * BQ + lax.broadcasted_iota(jnp.int32, (BQ, KVB), 0)
            cols = j * KVB + lax.broadcasted_iota(jnp.int32, (BQ, KVB), 1)
            ok = rows >= cols
            if j == NKVB - 1:
                ok = ok & (cols < T2)
            s = jnp.where(ok, s, NEG)
        m_prev = m_s[...]
        rm = jnp.max(s, axis=1, keepdims=True)
        m_new = jnp.maximum(m_prev, jnp.broadcast_to(rm, (BQ, D)))
        alpha = jnp.exp(m_prev - m_new)
        p = jnp.exp(s - jnp.broadcast_to(m_new[:, :1], (BQ, KVB)))
        l_s[...] = (l_s[...] * alpha
                    + jnp.broadcast_to(jnp.sum(p, axis=1, keepdims=True), (BQ, D)))
        vb = v_ref[0, j * KVB:(j + 1) * KVB, :]
        acc_s[...] = acc_s[...] * alpha + _dot(p.astype(bf16), vb)
        m_s[...] = m_new

    for j in range(NKVB):
        @pl.when(j * QPC + QPC <= qi)
        def _(j=j):
            _chunk(j, masked=False)

        @pl.when((j * QPC <= qi) & (qi < j * QPC + QPC))
        def _(j=j):
            _chunk(j, masked=True)
    o_ref[0] = (acc_s[...] / l_s[...]).astype(bf16)


def _attn(qf, kf, vf):
    return pl.pallas_call(
        _attn_body,
        grid=(B, NH, NQB),
        in_specs=[
            pl.BlockSpec((1, BQ, D), lambda b, h, qi: (b, qi, h)),
            pl.BlockSpec((1, T2P, D), lambda b, h, qi: (b, 0, h // G)),
            pl.BlockSpec((1, T2P, D), lambda b, h, qi: (b, 0, h // G)),
        ],
        out_specs=pl.BlockSpec((1, BQ, D), lambda b, h, qi: (b, qi, h)),
        out_shape=jax.ShapeDtypeStruct((B, T2P, NH * D), bf16),
        scratch_shapes=[pltpu.VMEM((BQ, D), f32), pltpu.VMEM((BQ, D), f32),
                        pltpu.VMEM((BQ, D), f32)],
        compiler_params=pltpu.CompilerParams(
            dimension_semantics=("parallel", "arbitrary", "arbitrary")),
        name="attn",
    )(qf, kf, vf)


def _oproj_body(o_ref, wo_ref, r_ref, y_ref):
    y_ref[0] = r_ref[0] + _dot(o_ref[0], wo_ref[...])


def _oproj(o, wo_bf, x2p):
    return pl.pallas_call(
        _oproj_body,
        grid=(B, T2P // 128),
        in_specs=[
            pl.BlockSpec((1, 128, NH * D), lambda b, r: (b, r, 0)),
            pl.BlockSpec((NH * D, HD), lambda b, r: (0, 0)),
            pl.BlockSpec((1, 128, HD), lambda b, r: (b, r, 0)),
        ],
        out_specs=pl.BlockSpec((1, 128, HD), lambda b, r: (b, r, 0)),
        out_shape=jax.ShapeDtypeStruct((B, T2P, HD), f32),
        compiler_params=pltpu.CompilerParams(
            dimension_semantics=("parallel", "arbitrary")),
        name="oproj",
    )(o, wo_bf, x2p)


BR = 1152
BF = 256


def _mlp_body(y_ref, lw_ref, wg_ref, wu_ref, wd_ref, out_ref, xn_s):
    fi = pl.program_id(2)

    @pl.when(fi == 0)
    def _():
        yv = y_ref[...]
        xn_s[...] = _rms(yv, lw_ref[...]).astype(bf16)
        out_ref[...] = yv

    gv = _dot(xn_s[...], wg_ref[...])
    uv = _dot(xn_s[...], wu_ref[...])
    act = (_silu(gv) * uv).astype(bf16)
    out_ref[...] += _dot(act, wd_ref[...])


def _mlp(y2d, ln2w, wg_bf, wu_bf, wd_bf):
    rows = B * T2P
    return pl.pallas_call(
        _mlp_body,
        grid=(2, rows // BR // 2, FF // BF),
        in_specs=[
            pl.BlockSpec((BR, HD), lambda cb, r, fi: (cb * (rows // BR // 2) + r, 0)),
            pl.BlockSpec((1, HD), lambda cb, r, fi: (0, 0)),
            pl.BlockSpec((HD, BF), lambda cb, r, fi: (0, fi)),
            pl.BlockSpec((HD, BF), lambda cb, r, fi: (0, fi)),
            pl.BlockSpec((BF, HD), lambda cb, r, fi: (fi, 0)),
        ],
        out_specs=pl.BlockSpec((BR, HD), lambda cb, r, fi: (cb * (rows // BR // 2) + r, 0)),
        out_shape=jax.ShapeDtypeStruct((rows, HD), f32),
        scratch_shapes=[pltpu.VMEM((BR, HD), bf16)],
        compiler_params=pltpu.CompilerParams(
            dimension_semantics=("parallel", "arbitrary", "arbitrary"),
            vmem_limit_bytes=56 * 1024 * 1024),
        name="mlp",
    )(y2d, ln2w, wg_bf, wu_bf, wd_bf)


def kernel(x, cos, sin, mem_idx, params):
    p = params
    ln1w = p["ln1_w"].reshape(1, HD)
    ln2w = p["ln2_w"].reshape(1, HD)

    xm = x[:, SINK:T - WIN]
    w_in_pad = jnp.pad(p["m_in_w"], ((0, 0), (0, CINP - CIN))).astype(bf16)
    zx = _inproj(xm, w_in_pad, ln1w)

    w16 = p["m_in_w"][:, DI + CONV:].astype(bf16)
    dtb_col = p["m_dt_bias"].reshape(NH, 1)
    aexp_col = jnp.exp(p["m_A_log"]).reshape(NH, 1)
    dt_t, la_t = _dtproj(x, ln1w, w16, dtb_col, aexp_col)

    wconv = p["m_conv_w"][:, 0, :].T
    convb = p["m_conv_b"].reshape(1, CONV)
    xbca = _conv(zx, wconv, convb)

    y_ssd = _ssd(xbca, dt_t, la_t, p["m_D"])

    ym, zm = _gather(mem_idx.astype(jnp.int32), y_ssd, zx)

    cosm = cos[0, SINK:SINK + M]
    sinm = sin[0, SINK:SINK + M]
    cos8 = jnp.tile(cosm, (1, KVH))
    sin8 = jnp.tile(sinm, (1, KVH))
    wkv = jnp.concatenate([p["wk"], p["wv"]], axis=1).astype(bf16)
    bkv = jnp.concatenate([p["bk"], p["bv"]]).reshape(1, 2 * KVH * D)
    mk, mv = _memfin(ym, zm, p["m_norm_w"].reshape(1, DI),
                     p["m_out_w"].astype(bf16), wkv, bkv, cos8, sin8)

    zpad = jnp.zeros((B, T2P - T2, HD), f32)
    x2p = jnp.concatenate([x[:, :SINK + M], x[:, T - WIN:], zpad], axis=1)
    zc = jnp.zeros((B, T2P - T2, D), f32)
    cos2p = jnp.concatenate([cos[:, :SINK + M], cos[:, T - WIN:], zc], axis=1)
    sin2p = jnp.concatenate([sin[:, :SINK + M], sin[:, T - WIN:], zc], axis=1)
    wqkv = jnp.concatenate([p["wq"], p["wk"], p["wv"]], axis=1).astype(bf16)
    bqkv = jnp.concatenate([p["bq"], p["bk"], p["bv"]]).reshape(1, -1)
    qf, k0, v0 = _qkv(x2p, ln1w, wqkv, bqkv, cos2p, sin2p)

    zkv = jnp.zeros((B, T2P - T2, KVH * D), bf16)
    kf = jnp.concatenate([k0[:, :SINK], mk, k0[:, SINK + M:T2], zkv], axis=1)
    vf = jnp.concatenate([v0[:, :SINK], mv, v0[:, SINK + M:T2], zkv], axis=1)

    o = _attn(qf, kf, vf)
    y1 = _oproj(o, p["wo"].astype(bf16), x2p)

    out = _mlp(y1.reshape(B * T2P, HD), ln2w,
               p["wg"].astype(bf16), p["wu"].astype(bf16), p["wd"].astype(bf16))
    return out.reshape(B, T2P, HD)[:, :T2]


# speedup vs baseline: 1.0333x; 1.0333x over previous
import functools
import math

import jax
import jax.numpy as jnp
from jax import lax
from jax.experimental import pallas as pl
from jax.experimental.pallas import tpu as pltpu


B = 2
T = 4096
HD = 2048
NH = 16
KVH = 8
D = 128
SINK = 128
WIN = 2048
M = 64
G = 2
N = 128
DI = HD
K = 4
CONV = DI + 2 * G * N
FF = 8192
E = T - WIN - SINK
T2 = SINK + M + WIN
T2P = 2304
NKV = 2 * G * N
CIN = DI + CONV + NH
CINP = 4864
L = 128
NC = E // L
EPS = 1e-6
SCALE = 1.0 / math.sqrt(D)
NEG = -1e30

f32 = jnp.float32
bf16 = jnp.bfloat16


def _rms(xv, w):
    ms = jnp.mean(xv * xv, axis=-1, keepdims=True)
    return xv * lax.rsqrt(ms + EPS) * w


def _softplus(xv):
    return jnp.maximum(xv, 0.0) + jnp.log1p(jnp.exp(-jnp.abs(xv)))


def _silu(xv):
    return xv * jax.nn.sigmoid(xv)


def _rope(xv, c, s, groups):
    parts = []
    for g in range(groups):
        a = xv[:, g * 128:g * 128 + 64]
        b = xv[:, g * 128 + 64:(g + 1) * 128]
        parts.append(-b)
        parts.append(a)
    rh = jnp.concatenate(parts, axis=1)
    return xv * c + rh * s


def _dot(a, b):
    return jnp.dot(a, b, preferred_element_type=f32)


def _dotg(a, b, dims):
    return lax.dot_general(a, b, (dims, ((), ())), preferred_element_type=f32)


def _inproj_body(x_ref, w_ref, lw_ref, zx_ref, xn_s):
    j = pl.program_id(2)

    @pl.when(j == 0)
    def _():
        xn = _rms(x_ref[0], lw_ref[...])
        xn_s[...] = xn.astype(bf16)

    zx_ref[:, 0, :] = _dot(xn_s[...], w_ref[...])


def _inproj(xm, w_in_bf, ln1w):
    return pl.pallas_call(
        _inproj_body,
        grid=(B, 2, CINP // 256),
        in_specs=[
            pl.BlockSpec((1, E // 2, HD), lambda b, p, j: (b, p, 0)),
            pl.BlockSpec((HD, 256), lambda b, p, j: (0, j)),
            pl.BlockSpec((1, HD), lambda b, p, j: (0, 0)),
        ],
        out_specs=pl.BlockSpec(
            (E // 2, 1, 256), lambda b, p, j: (p, 0, b * (CINP // 256) + j)),
        out_shape=jax.ShapeDtypeStruct((E, 1, B * CINP), f32),
        scratch_shapes=[pltpu.VMEM((E // 2, HD), bf16)],
        compiler_params=pltpu.CompilerParams(
            dimension_semantics=("parallel", "arbitrary", "arbitrary"),
            vmem_limit_bytes=48 * 1024 * 1024),
        name="inproj",
    )(xm, w_in_bf, ln1w)


def _dtproj_body(x_ref, lw_ref, w16_ref, dtb_ref, ae_ref, dt_ref, la_ref):
    xn = _rms(x_ref[0], lw_ref[...]).astype(bf16)
    raw = _dotg(w16_ref[...], xn, (((0,), (1,))))
    dt = _softplus(raw + dtb_ref[...])
    la = -ae_ref[...] * dt
    dt_ref[...] = dt.reshape(NH, 1, L)
    la_ref[...] = la.reshape(NH, 1, L)


def _dtproj(x, ln1w, w16_bf, dtb_col, aexp_col):
    return pl.pallas_call(
        _dtproj_body,
        grid=(B, NC),
        in_specs=[
            pl.BlockSpec((1, L, HD), lambda b, c: (b, c + 1, 0)),
            pl.BlockSpec((1, HD), lambda b, c: (0, 0)),
            pl.BlockSpec((HD, NH), lambda b, c: (0, 0)),
            pl.BlockSpec((NH, 1), lambda b, c: (0, 0)),
            pl.BlockSpec((NH, 1), lambda b, c: (0, 0)),
        ],
        out_specs=[
            pl.BlockSpec((NH, 1, L), lambda b, c: (b, 0, c)),
            pl.BlockSpec((NH, 1, L), lambda b, c: (b, 0, c)),
        ],
        out_shape=[
            jax.ShapeDtypeStruct((B * NH, 1, E), f32),
            jax.ShapeDtypeStruct((B * NH, 1, E), f32),
        ],
        compiler_params=pltpu.CompilerParams(
            dimension_semantics=("parallel", "arbitrary")),
        name="dtproj",
    )(x, ln1w, w16_bf, dtb_col, aexp_col)


def _conv_body(xbc_ref, wc_ref, cb_ref, out_ref):
    xv = xbc_ref[:, 0, :]
    acc = cb_ref[...] + wc_ref[3:4, :] * xv
    for k in range(3):
        sh = 3 - k
        shifted = jnp.concatenate(
            [jnp.zeros((sh, 128), f32), xv[:E - sh, :]], axis=0)
        acc = acc + wc_ref[k:k + 1, :] * shifted
    out_ref[:, 0, :] = _silu(acc)


def _conv(zx, wconv, convb):
    return pl.pallas_call(
        _conv_body,
        grid=(B, CONV // 128),
        in_specs=[
            pl.BlockSpec((E, 1, 128),
                         lambda b, j: (0, 0, b * (CINP // 128) + DI // 128 + j)),
            pl.BlockSpec((K, 128), lambda b, j: (0, j)),
            pl.BlockSpec((1, 128), lambda b, j: (0, j)),
        ],
        out_specs=pl.BlockSpec((E, 1, 128), lambda b, j: (0, 0, b * (CONV // 128) + j)),
        out_shape=jax.ShapeDtypeStruct((E, 1, B * CONV), f32),
        compiler_params=pltpu.CompilerParams(
            dimension_semantics=("parallel", "arbitrary")),
        name="conv",
    )(zx, wconv, convb)


def _ssd_body(xbc_ref, dt_ref, la_ref, mD_ref, y_ref, st_ref):
    c = pl.program_id(1)

    @pl.when(c == 0)
    def _():
        st_ref[...] = jnp.zeros_like(st_ref)

    la_all = la_ref[:, 0, :]
    dt_all = dt_ref[:, 0, :]
    s_all = la_all
    sh = 1
    while sh < L:
        s_all = s_all + jnp.concatenate(
            [jnp.zeros((NH, sh), f32), s_all[:, :L - sh]], axis=1)
        sh *= 2
    sT = jnp.transpose(s_all)
    dtT = jnp.transpose(dt_all)

    xbc = xbc_ref[:, 0, :]
    row_i = lax.broadcasted_iota(jnp.int32, (L, L), 0)
    col_i = lax.broadcasted_iota(jnp.int32, (L, L), 1)
    tri = row_i >= col_i

    for h in range(NH):
        g = h // (NH // G)
        xs = xbc[:, h * 128:(h + 1) * 128]
        Bm = xbc[:, DI + g * 128:DI + (g + 1) * 128]
        Cm = xbc[:, DI + G * 128 + g * 128:DI + G * 128 + (g + 1) * 128]
        s_row = s_all[h:h + 1, :]
        s_colb = jnp.broadcast_to(sT[:, h:h + 1], (L, L))
        dt_row = dt_all[h:h + 1, :]
        arg = jnp.where(tri, s_colb - jnp.broadcast_to(s_row, (L, L)), NEG)
        Wm = jnp.exp(arg) * jnp.broadcast_to(dt_row, (L, L))
        CB = _dotg(Cm.astype(bf16), Bm.astype(bf16), ((1,), (1,)))
        scores = CB * Wm
        h0 = st_ref[h * 128:(h + 1) * 128, :]
        Ch0 = _dotg(Cm.astype(bf16), h0.astype(bf16), ((1,), (1,)))
        exp_s = jnp.exp(s_colb)
        y_h = (_dotg(scores.astype(bf16), xs.astype(bf16), ((1,), (0,)))
               + Ch0 * exp_s + mD_ref[h] * xs)
        y_ref[:, 0, h * 128:(h + 1) * 128] = y_h
        s_last = sT[L - 1, h]
        w_col = jnp.exp(s_last - sT[:, h:h + 1]) * dtT[:, h:h + 1]
        xs_sc = xs * jnp.broadcast_to(w_col, (L, L))
        st_ref[h * 128:(h + 1) * 128, :] = (
            h0 * jnp.exp(s_last)
            + _dotg(xs_sc.astype(bf16), Bm.astype(bf16), ((0,), (0,))))


def _ssd(xbca, dt_t, la_t, mD):
    return pl.pallas_call(
        _ssd_body,
        grid=(B, NC),
        in_specs=[
            pl.BlockSpec((L, 1, CONV), lambda b, c: (c, 0, b)),
            pl.BlockSpec((NH, 1, L), lambda b, c: (b, 0, c)),
            pl.BlockSpec((NH, 1, L), lambda b, c: (b, 0, c)),
            pl.BlockSpec(memory_space=pltpu.SMEM),
        ],
        out_specs=pl.BlockSpec((L, 1, DI), lambda b, c: (c, 0, b)),
        out_shape=jax.ShapeDtypeStruct((E, 1, B * DI), f32),
        scratch_shapes=[pltpu.VMEM((NH * D, N), f32)],
        compiler_params=pltpu.CompilerParams(
            dimension_semantics=("parallel", "arbitrary")),
        name="ssd",
    )(xbca, dt_t, la_t, mD)


def _gather_body(idx_ref, y_ref, zx_ref, ym_ref, zm_ref):
    ym_ref[...] = y_ref[...]
    zm_ref[...] = zx_ref[...]


def _gather(mem_idx, y, zx):
    return pl.pallas_call(
        _gather_body,
        grid_spec=pltpu.PrefetchScalarGridSpec(
            num_scalar_prefetch=1,
            grid=(M,),
            in_specs=[
                pl.BlockSpec((1, 1, B * DI), lambda m, idx: (idx[m], 0, 0)),
                pl.BlockSpec((1, 1, B * CINP), lambda m, idx: (idx[m], 0, 0)),
            ],
            out_specs=[
                pl.BlockSpec((1, 1, B * DI), lambda m, idx: (m, 0, 0)),
                pl.BlockSpec((1, 1, B * CINP), lambda m, idx: (m, 0, 0)),
            ],
        ),
        out_shape=[
            jax.ShapeDtypeStruct((M, 1, B * DI), f32),
            jax.ShapeDtypeStruct((M, 1, B * CINP), f32),
        ],
        name="gather",
    )(mem_idx, y, zx)


def _memfin_body(ym_ref, zm_ref, nw_ref, mow_ref, wkv_ref, bkv_ref,
                 c8_ref, s8_ref, mk_ref, mv_ref):
    y64 = ym_ref[:, 0, :]
    z64 = zm_ref[:, 0, :DI]
    gt = y64 * _silu(z64)
    nr = _rms(gt, nw_ref[...]).astype(bf16)
    act = _dot(nr, mow_ref[...]).astype(bf16)
    kv = _dot(act, wkv_ref[...]) + bkv_ref[...]
    kpart = kv[:, :KVH * D]
    mk = _rope(kpart, c8_ref[...], s8_ref[...], KVH)
    mk_ref[0] = mk.astype(bf16)
    mv_ref[0] = kv[:, KVH * D:].astype(bf16)


def _memfin(ym, zm, mnorm, mow_bf, wkv_bf, bkv, cos8, sin8):
    return pl.pallas_call(
        _memfin_body,
        grid=(B,),
        in_specs=[
            pl.BlockSpec((M, 1, DI), lambda b: (0, 0, b)),
            pl.BlockSpec((M, 1, CINP), lambda b: (0, 0, b)),
            pl.BlockSpec((1, DI), lambda b: (0, 0)),
            pl.BlockSpec((DI, DI), lambda b: (0, 0)),
            pl.BlockSpec((DI, 2 * KVH * D), lambda b: (0, 0)),
            pl.BlockSpec((1, 2 * KVH * D), lambda b: (0, 0)),
            pl.BlockSpec((M, KVH * D), lambda b: (0, 0)),
            pl.BlockSpec((M, KVH * D), lambda b: (0, 0)),
        ],
        out_specs=[
            pl.BlockSpec((1, M, KVH * D), lambda b: (b, 0, 0)),
            pl.BlockSpec((1, M, KVH * D), lambda b: (b, 0, 0)),
        ],
        out_shape=[
            jax.ShapeDtypeStruct((B, M, KVH * D), bf16),
            jax.ShapeDtypeStruct((B, M, KVH * D), bf16),
        ],
        compiler_params=pltpu.CompilerParams(
            dimension_semantics=("parallel",),
            vmem_limit_bytes=48 * 1024 * 1024),
        name="memfin",
    )(ym, zm, mnorm, mow_bf, wkv_bf, bkv, cos8, sin8)


def _qkv_body(x_ref, lw_ref, w_ref, bias_ref, c_ref, s_ref,
              q_ref, k_ref, v_ref):
    xn = _rms(x_ref[0], lw_ref[...]).astype(bf16)
    qkv = _dot(xn, w_ref[...]) + bias_ref[...]
    q = qkv[:, :NH * D]
    k = qkv[:, NH * D:NH * D + KVH * D]
    v = qkv[:, NH * D + KVH * D:]
    cos1 = c_ref[0]
    sin1 = s_ref[0]
    c16 = jnp.concatenate([cos1] * NH, axis=1)
    s16 = jnp.concatenate([sin1] * NH, axis=1)
    c8 = jnp.concatenate([cos1] * KVH, axis=1)
    s8 = jnp.concatenate([sin1] * KVH, axis=1)
    q_ref[0] = (_rope(q, c16, s16, NH) * SCALE).astype(bf16)
    k_ref[0] = _rope(k, c8, s8, KVH).astype(bf16)
    v_ref[0] = v.astype(bf16)


def _qkv(x2p, ln1w, wqkv_bf, bqkv, cos2p, sin2p):
    return pl.pallas_call(
        _qkv_body,
        grid=(B, T2P // 128),
        in_specs=[
            pl.BlockSpec((1, 128, HD), lambda b, r: (b, r, 0)),
            pl.BlockSpec((1, HD), lambda b, r: (0, 0)),
            pl.BlockSpec((HD, NH * D + 2 * KVH * D), lambda b, r: (0, 0)),
            pl.BlockSpec((1, NH * D + 2 * KVH * D), lambda b, r: (0, 0)),
            pl.BlockSpec((1, 128, D), lambda b, r: (b, r, 0)),
            pl.BlockSpec((1, 128, D), lambda b, r: (b, r, 0)),
        ],
        out_specs=[
            pl.BlockSpec((1, 128, NH * D), lambda b, r: (b, r, 0)),
            pl.BlockSpec((1, 128, KVH * D), lambda b, r: (b, r, 0)),
            pl.BlockSpec((1, 128, KVH * D), lambda b, r: (b, r, 0)),
        ],
        out_shape=[
            jax.ShapeDtypeStruct((B, T2P, NH * D), bf16),
            jax.ShapeDtypeStruct((B, T2P, KVH * D), bf16),
            jax.ShapeDtypeStruct((B, T2P, KVH * D), bf16),
        ],
        compiler_params=pltpu.CompilerParams(
            dimension_semantics=("parallel", "arbitrary"),
            vmem_limit_bytes=56 * 1024 * 1024),
        name="qkv",
    )(x2p, ln1w, wqkv_bf, bqkv, cos2p, sin2p)


BQ = 256
NQB = T2P // BQ


KVB = 768
NKVB = T2P // KVB
QPC = KVB // BQ


def _attn_body(q_ref, k_ref, v_ref, o_ref, m_s, l_s, acc_s):
    qi = pl.program_id(2)
    m_s[...] = jnp.full_like(m_s, NEG)
    l_s[...] = jnp.zeros_like(l_s)
    acc_s[...] = jnp.zeros_like(acc_s)
    qb = q_ref[0]

    def _chunk(j, masked):
        kb = k_ref[0, j * KVB:(j + 1) * KVB, :]
        s = _dotg(qb, kb, ((1,), (1,)))
        if masked:
            rows = qi * BQ + lax.broadcasted_iota(jnp.int32, (BQ, KVB), 0)
            cols = j * KVB + lax.broadcasted_iota(jnp.int32, (BQ, KVB), 1)
            ok = rows >= cols
            if j == NKVB - 1:
                ok = ok & (cols < T2)
            s = jnp.where(ok, s, NEG)
        m_prev = m_s[...]
        rm = jnp.max(s, axis=1, keepdims=True)
        m_new = jnp.maximum(m_prev, jnp.broadcast_to(rm, (BQ, D)))
        alpha = jnp.exp(m_prev - m_new)
        p = jnp.exp(s - jnp.broadcast_to(m_new[:, :1], (BQ, KVB)))
        l_s[...] = (l_s[...] * alpha
                    + jnp.broadcast_to(jnp.sum(p, axis=1, keepdims=True), (BQ, D)))
        vb = v_ref[0, j * KVB:(j + 1) * KVB, :]
        acc_s[...] = acc_s[...] * alpha + _dot(p.astype(bf16), vb)
        m_s[...] = m_new

    for j in range(NKVB):
        @pl.when(j * QPC + QPC <= qi)
        def _(j=j):
            _chunk(j, masked=False)

        @pl.when((j * QPC <= qi) & (qi < j * QPC + QPC))
        def _(j=j):
            _chunk(j, masked=True)
    o_ref[0] = (acc_s[...] / l_s[...]).astype(bf16)


def _attn(qf, kf, vf):
    return pl.pallas_call(
        _attn_body,
        grid=(B, NH, NQB),
        in_specs=[
            pl.BlockSpec((1, BQ, D), lambda b, h, qi: (b, qi, h)),
            pl.BlockSpec((1, T2P, D), lambda b, h, qi: (b, 0, h // G)),
            pl.BlockSpec((1, T2P, D), lambda b, h, qi: (b, 0, h // G)),
        ],
        out_specs=pl.BlockSpec((1, BQ, D), lambda b, h, qi: (b, qi, h)),
        out_shape=jax.ShapeDtypeStruct((B, T2P, NH * D), bf16),
        scratch_shapes=[pltpu.VMEM((BQ, D), f32), pltpu.VMEM((BQ, D), f32),
                        pltpu.VMEM((BQ, D), f32)],
        compiler_params=pltpu.CompilerParams(
            dimension_semantics=("parallel", "arbitrary", "arbitrary")),
        name="attn",
    )(qf, kf, vf)


def _oproj_body(o_ref, wo_ref, r_ref, y_ref):
    y_ref[0] = r_ref[0] + _dot(o_ref[0], wo_ref[...])


def _oproj(o, wo_bf, x2p):
    return pl.pallas_call(
        _oproj_body,
        grid=(B, T2P // 128),
        in_specs=[
            pl.BlockSpec((1, 128, NH * D), lambda b, r: (b, r, 0)),
            pl.BlockSpec((NH * D, HD), lambda b, r: (0, 0)),
            pl.BlockSpec((1, 128, HD), lambda b, r: (b, r, 0)),
        ],
        out_specs=pl.BlockSpec((1, 128, HD), lambda b, r: (b, r, 0)),
        out_shape=jax.ShapeDtypeStruct((B, T2P, HD), f32),
        compiler_params=pltpu.CompilerParams(
            dimension_semantics=("parallel", "arbitrary")),
        name="oproj",
    )(o, wo_bf, x2p)


BR = 1152
BF = 256


def _mlp_body(y_ref, lw_ref, wg_ref, wu_ref, wd_ref, out_ref, xn_s):
    fi = pl.program_id(2)

    @pl.when(fi == 0)
    def _():
        yv = y_ref[...]
        xn_s[...] = _rms(yv, lw_ref[...]).astype(bf16)
        out_ref[...] = yv

    gv = _dot(xn_s[...], wg_ref[...])
    uv = _dot(xn_s[...], wu_ref[...])
    act = (_silu(gv) * uv).astype(bf16)
    out_ref[...] += _dot(act, wd_ref[...])


def _mlp(y2d, ln2w, wg_bf, wu_bf, wd_bf):
    rows = B * T2P
    return pl.pallas_call(
        _mlp_body,
        grid=(2, rows // BR // 2, FF // BF),
        in_specs=[
            pl.BlockSpec((BR, HD), lambda cb, r, fi: (cb * (rows // BR // 2) + r, 0)),
            pl.BlockSpec((1, HD), lambda cb, r, fi: (0, 0)),
            pl.BlockSpec((HD, BF), lambda cb, r, fi: (0, fi)),
            pl.BlockSpec((HD, BF), lambda cb, r, fi: (0, fi)),
            pl.BlockSpec((BF, HD), lambda cb, r, fi: (fi, 0)),
        ],
        out_specs=pl.BlockSpec((BR, HD), lambda cb, r, fi: (cb * (rows // BR // 2) + r, 0)),
        out_shape=jax.ShapeDtypeStruct((rows, HD), f32),
        scratch_shapes=[pltpu.VMEM((BR, HD), bf16)],
        compiler_params=pltpu.CompilerParams(
            dimension_semantics=("parallel", "arbitrary", "arbitrary"),
            vmem_limit_bytes=56 * 1024 * 1024),
        name="mlp",
    )(y2d, ln2w, wg_bf, wu_bf, wd_bf)


def kernel(x, cos, sin, mem_idx, params):
    p = params
    ln1w = p["ln1_w"].reshape(1, HD)
    ln2w = p["ln2_w"].reshape(1, HD)

    xm = x[:, SINK:T - WIN]
    w_in_pad = jnp.pad(p["m_in_w"], ((0, 0), (0, CINP - CIN))).astype(bf16)
    zx = _inproj(xm, w_in_pad, ln1w)

    w16 = p["m_in_w"][:, DI + CONV:].astype(bf16)
    dtb_col = p["m_dt_bias"].reshape(NH, 1)
    aexp_col = jnp.exp(p["m_A_log"]).reshape(NH, 1)
    dt_t, la_t = _dtproj(x, ln1w, w16, dtb_col, aexp_col)

    wconv = p["m_conv_w"][:, 0, :].T
    convb = p["m_conv_b"].reshape(1, CONV)
    xbca = _conv(zx, wconv, convb)

    y_ssd = _ssd(xbca, dt_t, la_t, p["m_D"])

    ym, zm = _gather(mem_idx.astype(jnp.int32), y_ssd, zx)

    cosm = cos[0, SINK:SINK + M]
    sinm = sin[0, SINK:SINK + M]
    cos8 = jnp.tile(cosm, (1, KVH))
    sin8 = jnp.tile(sinm, (1, KVH))
    wkv = jnp.concatenate([p["wk"], p["wv"]], axis=1).astype(bf16)
    bkv = jnp.concatenate([p["bk"], p["bv"]]).reshape(1, 2 * KVH * D)
    mk, mv = _memfin(ym, zm, p["m_norm_w"].reshape(1, DI),
                     p["m_out_w"].astype(bf16), wkv, bkv, cos8, sin8)

    zpad = jnp.zeros((B, T2P - T2, HD), f32)
    x2p = jnp.concatenate([x[:, :SINK + M], x[:, T - WIN:], zpad], axis=1)
    zc = jnp.zeros((B, T2P - T2, D), f32)
    cos2p = jnp.concatenate([cos[:, :SINK + M], cos[:, T - WIN:], zc], axis=1)
    sin2p = jnp.concatenate([sin[:, :SINK + M], sin[:, T - WIN:], zc], axis=1)
    wqkv = jnp.concatenate([p["wq"], p["wk"], p["wv"]], axis=1).astype(bf16)
    bqkv = jnp.concatenate([p["bq"], p["bk"], p["bv"]]).reshape(1, -1)
    qf, k0, v0 = _qkv(x2p, ln1w, wqkv, bqkv, cos2p, sin2p)

    zkv = jnp.zeros((B, T2P - T2, KVH * D), bf16)
    kf = jnp.concatenate([k0[:, :SINK], mk, k0[:, SINK + M:T2], zkv], axis=1)
    vf = jnp.concatenate([v0[:, :SINK], mv, v0[:, SINK + M:T2], zkv], axis=1)

    o = _attn(qf, kf, vf)
    y1 = _oproj(o, p["wo"].astype(bf16), x2p)

    out = _mlp(y1.reshape(B * T2P, HD), ln2w,
               p["wg"].astype(bf16), p["wu"].astype(bf16), p["wd"].astype(bf16))
    return out.reshape(B, T2P, HD)[:, :T2]
